# Initial kernel scaffold; baseline (speedup 1.0000x reference)
import jax
import jax.numpy as jnp
from jax.experimental import pallas as pl


def kernel(x, edge_index, edge_d, W1, b1, Wq, bq, Wk, bk, Wv, bv, Wo, bo, W2, b2, Wt, bt):
    raise NotImplementedError("write your pallas kernel here")


# speedup vs baseline: 1.1489x; 1.1489x over previous
import math

import jax
import jax.numpy as jnp
from jax.experimental import pallas as pl

N = 10000
E = 160000
HID = 256
OUT = 256
TOPK = int(math.log(N))


def _identity_body(x_ref, o_ref):
    o_ref[...] = x_ref[...]


def kernel(x, edge_index, edge_d, W1, b1, Wq, bq, Wk, bk, Wv, bv, Wo, bo, W2, b2, Wt, bt):
    src = edge_index[0]
    dst = edge_index[1]

    deg_out = jnp.zeros((N,), jnp.float32).at[src].add(1.0)
    deg_in = jnp.zeros((N,), jnp.float32).at[dst].add(1.0)
    has_iso = jnp.min(deg_in) < 0.5
    degoi = jnp.maximum(deg_out, 1.0) ** -0.5
    degii = jnp.maximum(deg_in, 1.0) ** -0.5

    xn = x * degoi[:, None]
    agg1 = jnp.zeros((N, x.shape[1]), jnp.float32).at[dst].add(xn[src])
    h1 = jax.nn.relu((agg1 * degii[:, None]) @ W1 + b1)

    q = h1 @ Wq + bq
    k = h1 @ Wk + bk
    v = h1 @ Wv + bv
    qf = jnp.fft.rfft(q.T, axis=-1)
    kf = jnp.fft.rfft(k.T, axis=-1)
    spec = jnp.sum(qf * jnp.conj(kf), axis=0)
    mean_value = jnp.fft.irfft(spec, n=N) * (1.0 / HID)
    weights, delay = jax.lax.top_k(mean_value, TOPK)
    tmp_corr = jax.nn.softmax(weights)
    v2 = jnp.concatenate([v, v], axis=0)
    acc = jnp.zeros_like(v)
    for i in range(TOPK):
        acc = acc + tmp_corr[i] * jax.lax.dynamic_slice_in_dim(v2, delay[i], N, axis=0)
    h_ac = acc @ Wo + bo

    h2n = h_ac * degoi[:, None]
    agg2 = jnp.zeros((N, HID), jnp.float32).at[dst].add(h2n[src])
    h3 = jax.nn.relu((agg2 * degii[:, None]) @ W2 + b2)

    g = h3 @ Wt
    msg = edge_d[:, None] * (g[src] + g[dst])
    m = jnp.max(msg, axis=0) + bt
    out = jnp.where(has_iso, jnp.maximum(m, 0.0), m)

    out = pl.pallas_call(
        _identity_body,
        out_shape=jax.ShapeDtypeStruct((1, OUT), jnp.float32),
    )(out[None, :])
    return out


# speedup vs baseline: 1.2424x; 1.0814x over previous
import functools
import math

import jax
import jax.numpy as jnp
from jax import lax
from jax.experimental import pallas as pl
from jax.experimental.pallas import tpu as pltpu
from jax.experimental.pallas import tpu_sc as plsc

N = 10000
E = 160000
INF = 128
HID = 256
TOPK = int(math.log(N))
NC, NS = 2, 16
NPAD = 10240
RPS = N // NS
EK_F, FS_CH = 80, 125
EK_E, ES_CH = 40, 125
LW = 128
DAG_T = 400
ER_C = 2000
HP = jax.lax.Precision.HIGHEST

_MESH = dict(core_axis_name="c", subcore_axis_name="s")


def _sc_degrees(se_f, ones_rows, zdeg):
    @functools.partial(
        pl.kernel,
        mesh=plsc.VectorSubcoreMesh(**_MESH),
        out_type=jax.ShapeDtypeStruct((NC, NPAD, LW), jnp.float32),
        scratch_types=[
            pltpu.VMEM((FS_CH, EK_F), jnp.int32),
            pltpu.VMEM((EK_F, LW), jnp.float32),
            pltpu.VMEM_SHARED((NPAD, LW), jnp.float32),
            pltpu.SemaphoreType.DMA,
        ],
    )
    def k(se_hbm, ones_hbm, z_hbm, out_hbm, idx_v, ones_v, acc_sh, sem):
        cid = lax.axis_index("c")
        sid = lax.axis_index("s")
        pltpu.sync_copy(se_hbm.at[cid, sid], idx_v)
        pltpu.sync_copy(ones_hbm, ones_v)

        @pl.when(sid == 0)
        def _():
            pltpu.sync_copy(z_hbm, acc_sh)

        plsc.subcore_barrier()

        def body(j, carry):
            pltpu.sync_copy(ones_v, acc_sh.at[idx_v.at[j]], add=True)
            return carry

        lax.fori_loop(0, FS_CH, body, 0)
        plsc.subcore_barrier()
        sl = pl.ds(sid * (NPAD // NS), NPAD // NS)
        pltpu.sync_copy(acc_sh.at[sl], out_hbm.at[cid].at[sl])

    return k(se_f, ones_rows, zdeg)


def _sc_edge_scatter_fsplit(table_h, se_f, zrows):
    @functools.partial(
        pl.kernel,
        mesh=plsc.VectorSubcoreMesh(**_MESH),
        out_type=jax.ShapeDtypeStruct((NC, NPAD, LW), jnp.float32),
        scratch_types=[
            pltpu.VMEM((FS_CH, EK_F), jnp.int32),
            pltpu.VMEM((FS_CH, EK_F), jnp.int32),
            pltpu.VMEM((EK_F, LW), jnp.float32),
            pltpu.VMEM_SHARED((NPAD, LW), jnp.float32),
            pltpu.SemaphoreType.DMA,
        ],
    )
    def k(tab_hbm, se_hbm, z_hbm, out_hbm, sidx_v, didx_v, rows_v, acc_sh, sem):
        cid = lax.axis_index("c")
        sid = lax.axis_index("s")
        pltpu.sync_copy(se_hbm.at[0, sid], sidx_v)
        pltpu.sync_copy(se_hbm.at[1, sid], didx_v)

        @pl.when(sid == 0)
        def _():
            pltpu.sync_copy(z_hbm, acc_sh)

        plsc.subcore_barrier()

        def body(j, carry):
            pltpu.async_copy(tab_hbm.at[cid].at[sidx_v.at[j]], rows_v, sem).wait()
            pltpu.sync_copy(rows_v, acc_sh.at[didx_v.at[j]], add=True)
            return carry

        lax.fori_loop(0, FS_CH, body, 0)
        plsc.subcore_barrier()
        sl = pl.ds(sid * (NPAD // NS), NPAD // NS)
        pltpu.sync_copy(acc_sh.at[sl], out_hbm.at[cid].at[sl])

    return k(table_h, se_f, zrows)


def _sc_edge_scatter_esplit(table_h, se_e, zrows):
    @functools.partial(
        pl.kernel,
        mesh=plsc.VectorSubcoreMesh(**_MESH),
        out_type=jax.ShapeDtypeStruct((NC, NPAD, LW), jnp.float32),
        scratch_types=[
            pltpu.VMEM((ES_CH, EK_E), jnp.int32),
            pltpu.VMEM((ES_CH, EK_E), jnp.int32),
            pltpu.VMEM((EK_E, LW), jnp.float32),
            pltpu.VMEM_SHARED((NPAD, LW), jnp.float32),
            pltpu.SemaphoreType.DMA,
        ],
    )
    def k(tab_hbm, se_hbm, z_hbm, out_hbm, sidx_v, didx_v, rows_v, acc_sh, sem):
        cid = lax.axis_index("c")
        sid = lax.axis_index("s")
        pltpu.sync_copy(se_hbm.at[0, cid, sid], sidx_v)
        pltpu.sync_copy(se_hbm.at[1, cid, sid], didx_v)

        @pl.when(sid == 0)
        def _():
            pltpu.sync_copy(z_hbm, acc_sh)

        plsc.subcore_barrier()

        def body(j, carry):
            pltpu.async_copy(tab_hbm.at[sidx_v.at[j]], rows_v, sem).wait()
            pltpu.sync_copy(rows_v, acc_sh.at[didx_v.at[j]], add=True)
            return carry

        lax.fori_loop(0, ES_CH, body, 0)
        plsc.subcore_barrier()
        sl = pl.ds(sid * (NPAD // NS), NPAD // NS)
        pltpu.sync_copy(acc_sh.at[sl], out_hbm.at[cid].at[sl])

    return k(table_h, se_e, zrows)


def _sc_er_gather(g_h, se_f):
    @functools.partial(
        pl.kernel,
        mesh=plsc.VectorSubcoreMesh(**_MESH),
        out_type=(
            jax.ShapeDtypeStruct((NC, E, LW), jnp.float32),
            jax.ShapeDtypeStruct((NC, E, LW), jnp.float32),
        ),
        scratch_types=[
            pltpu.VMEM((FS_CH, EK_F), jnp.int32),
            pltpu.VMEM((FS_CH, EK_F), jnp.int32),
            pltpu.VMEM((EK_F, LW), jnp.float32),
            pltpu.VMEM((EK_F, LW), jnp.float32),
            pltpu.SemaphoreType.DMA,
        ],
    )
    def k(g_hbm, se_hbm, gs_hbm, gd_hbm, sidx_v, didx_v, ra, rb, sem):
        cid = lax.axis_index("c")
        sid = lax.axis_index("s")
        pltpu.sync_copy(se_hbm.at[0, sid], sidx_v)
        pltpu.sync_copy(se_hbm.at[1, sid], didx_v)

        def body(j, carry):
            base = sid * (E // NS) + j * EK_F
            pltpu.async_copy(g_hbm.at[cid].at[sidx_v.at[j]], ra, sem).wait()
            pltpu.sync_copy(ra, gs_hbm.at[cid].at[pl.ds(base, EK_F)])
            pltpu.async_copy(g_hbm.at[cid].at[didx_v.at[j]], rb, sem).wait()
            pltpu.sync_copy(rb, gd_hbm.at[cid].at[pl.ds(base, EK_F)])
            return carry

        lax.fori_loop(0, FS_CH, body, 0)

    return k(g_h, se_f)


def _stage0_body(x_ref, dego_ref, degi_ref, xn_ref, oi_ref, ii_ref, fl_ref):
    oi = lax.rsqrt(jnp.maximum(dego_ref[...], 1.0))
    ii = lax.rsqrt(jnp.maximum(degi_ref[...], 1.0))
    xn_ref[...] = x_ref[...] * oi
    oi_ref[...] = oi
    ii_ref[...] = ii
    fl_ref[...] = jnp.where(jnp.min(degi_ref[...]) < 0.5, 1.0, 0.0).reshape(1, 1)


def _stage0(x, dego, degi):
    return pl.pallas_call(
        _stage0_body,
        out_shape=(
            jax.ShapeDtypeStruct((N, INF), jnp.float32),
            jax.ShapeDtypeStruct((N, 1), jnp.float32),
            jax.ShapeDtypeStruct((N, 1), jnp.float32),
            jax.ShapeDtypeStruct((1, 1), jnp.float32),
        ),
    )(x, dego, degi)


def _stagea_body(agg_ref, ii_ref, w1_ref, b1_ref, wq_ref, bq_ref, wk_ref,
                 bk_ref, wv_ref, bv_ref, q_ref, k_ref, v_ref):
    h = (agg_ref[0] + agg_ref[1]) * ii_ref[...]
    h1 = jax.nn.relu(jnp.dot(h, w1_ref[...], precision=HP) + b1_ref[...])
    q_ref[...] = jnp.dot(h1, wq_ref[...], precision=HP) + bq_ref[...]
    k_ref[...] = jnp.dot(h1, wk_ref[...], precision=HP) + bk_ref[...]
    v_ref[...] = jnp.dot(h1, wv_ref[...], precision=HP) + bv_ref[...]


def _stagea(agg1, degii, W1, b1, Wq, bq, Wk, bk, Wv, bv):
    R = 2000
    full = lambda i: (0, 0)
    blk = lambda i: (i, 0)
    return pl.pallas_call(
        _stagea_body,
        grid=(N // R,),
        in_specs=[
            pl.BlockSpec((NC, R, INF), lambda i: (0, i, 0)),
            pl.BlockSpec((R, 1), blk),
            pl.BlockSpec((INF, HID), full),
            pl.BlockSpec((1, HID), full),
            pl.BlockSpec((HID, HID), full),
            pl.BlockSpec((1, HID), full),
            pl.BlockSpec((HID, HID), full),
            pl.BlockSpec((1, HID), full),
            pl.BlockSpec((HID, HID), full),
            pl.BlockSpec((1, HID), full),
        ],
        out_specs=(
            pl.BlockSpec((R, HID), blk),
            pl.BlockSpec((R, HID), blk),
            pl.BlockSpec((R, HID), blk),
        ),
        out_shape=(
            jax.ShapeDtypeStruct((N, HID), jnp.float32),
            jax.ShapeDtypeStruct((N, HID), jnp.float32),
            jax.ShapeDtypeStruct((N, HID), jnp.float32),
        ),
    )(agg1, degii, W1, b1, Wq, bq, Wk, bk, Wv, bv)


def _topk_body(mv_ref, w_ref, d_ref):
    arr = mv_ref[...]
    row = lax.broadcasted_iota(jnp.int32, (80, 128), 0)
    col = lax.broadcasted_iota(jnp.int32, (80, 128), 1)
    flat = row * 128 + col
    vals, idxs = [], []
    for _ in range(TOPK):
        m = jnp.max(arr)
        idx = jnp.min(jnp.where(arr >= m, flat, jnp.int32(2 ** 30)))
        vals.append(m)
        idxs.append(idx)
        arr = jnp.where(flat == idx, -1e30, arr)
    m9 = vals[0]
    es = [jnp.exp(vv - m9) for vv in vals]
    s = es[0]
    for e in es[1:]:
        s = s + e
    lane = lax.broadcasted_iota(jnp.int32, (1, 128), 1)
    w = jnp.zeros((1, 128), jnp.float32)
    dl = jnp.zeros((1, 128), jnp.int32)
    for i in range(TOPK):
        w = jnp.where(lane == i, es[i] / s, w)
        dl = jnp.where(lane == i, idxs[i], dl)
    w_ref[...] = w
    d_ref[...] = dl


def _topk(mvp):
    return pl.pallas_call(
        _topk_body,
        out_shape=(
            jax.ShapeDtypeStruct((1, 128), jnp.float32),
            jax.ShapeDtypeStruct((1, 128), jnp.int32),
        ),
    )(mvp)


def _dagg_body(d_sref, w_sref, v2_ref, oi_ref, wo_ref, bo_ref, out_ref):
    base = pl.program_id(0) * DAG_T

    def shifted(i):
        s = base + d_sref[i]
        a = (s // 8) * 8
        r = s - a
        blk = v2_ref[pl.ds(a, DAG_T + 8), :]
        return pltpu.roll(blk, jnp.mod(-r, DAG_T + 8), 0)[:DAG_T]

    acc = w_sref[0] * shifted(0)
    for i in range(1, TOPK):
        acc = acc + w_sref[i] * shifted(i)
    h = jnp.dot(acc, wo_ref[...], precision=HP) + bo_ref[...]
    out_ref[...] = h * oi_ref[...]


def _delayagg(delays9, w9, v2, degoi, Wo, bo):
    grid_spec = pltpu.PrefetchScalarGridSpec(
        num_scalar_prefetch=2,
        grid=(N // DAG_T,),
        in_specs=[
            pl.BlockSpec((2 * N, HID), lambda i, d, w: (0, 0)),
            pl.BlockSpec((DAG_T, 1), lambda i, d, w: (i, 0)),
            pl.BlockSpec((HID, HID), lambda i, d, w: (0, 0)),
            pl.BlockSpec((1, HID), lambda i, d, w: (0, 0)),
        ],
        out_specs=pl.BlockSpec((DAG_T, HID), lambda i, d, w: (i, 0)),
    )
    return pl.pallas_call(
        _dagg_body,
        grid_spec=grid_spec,
        out_shape=jax.ShapeDtypeStruct((N, HID), jnp.float32),
    )(delays9, w9, v2, degoi, Wo, bo)


def _stageb_body(agg_ref, ii_ref, w2_ref, b2_ref, wt_ref, g_ref):
    h = agg_ref[...] * ii_ref[...]
    h3 = jax.nn.relu(jnp.dot(h, w2_ref[...], precision=HP) + b2_ref[...])
    g_ref[...] = jnp.dot(h3, wt_ref[...], precision=HP)


def _stageb(agg2, degii, W2, b2, Wt):
    R = 2000
    full = lambda i: (0, 0)
    blk = lambda i: (i, 0)
    return pl.pallas_call(
        _stageb_body,
        grid=(N // R,),
        in_specs=[
            pl.BlockSpec((R, HID), blk),
            pl.BlockSpec((R, 1), blk),
            pl.BlockSpec((HID, HID), full),
            pl.BlockSpec((1, HID), full),
            pl.BlockSpec((HID, HID), full),
        ],
        out_specs=pl.BlockSpec((R, HID), blk),
        out_shape=jax.ShapeDtypeStruct((N, HID), jnp.float32),
    )(agg2, degii, W2, b2, Wt)


_ER_NCH = E // ER_C


def _ermax_body(gs_ref, gd_ref, d_ref, bt_ref, fl_ref, out_ref):
    j = pl.program_id(0)

    @pl.when(j == 0)
    def _():
        out_ref[...] = jnp.full((NC, 128), -jnp.inf, jnp.float32)

    msg = d_ref[0] * (gs_ref[...] + gd_ref[...])
    cur = jnp.maximum(out_ref[...], jnp.max(msg, axis=1))

    @pl.when(j < _ER_NCH - 1)
    def _():
        out_ref[...] = cur

    @pl.when(j == _ER_NCH - 1)
    def _():
        mm = cur + bt_ref[...]
        out_ref[...] = jnp.where(fl_ref[0, 0] > 0.5, jnp.maximum(mm, 0.0), mm)


def _ermax(gs, gd, d4, btr, flag):
    return pl.pallas_call(
        _ermax_body,
        grid=(_ER_NCH,),
        in_specs=[
            pl.BlockSpec((NC, ER_C, INF), lambda j: (0, j, 0)),
            pl.BlockSpec((NC, ER_C, INF), lambda j: (0, j, 0)),
            pl.BlockSpec((1, ER_C, 1), lambda j: (j, 0, 0)),
            pl.BlockSpec((NC, 128), lambda j: (0, 0)),
            pl.BlockSpec((1, 1), lambda j: (0, 0)),
        ],
        out_specs=pl.BlockSpec((NC, 128), lambda j: (0, 0)),
        out_shape=jax.ShapeDtypeStruct((NC, 128), jnp.float32),
    )(gs, gd, d4, btr, flag)


def kernel(x, edge_index, edge_d, W1, b1, Wq, bq, Wk, bk, Wv, bv, Wo, bo, W2, b2, Wt, bt):
    se = jnp.stack([edge_index[0], edge_index[1]])
    se_f = se.reshape(2, NS, FS_CH, EK_F)
    se_e = se.reshape(2, NC, NS, ES_CH, EK_E)
    ones_rows = jnp.ones((EK_F, LW), jnp.float32)
    zdeg = jnp.zeros((NPAD, LW), jnp.float32)

    degs = _sc_degrees(se_f, ones_rows, zdeg)
    dego = degs[0, :N, 0].reshape(N, 1)
    degi = degs[1, :N, 0].reshape(N, 1)

    xn, degoi, degii, flag = _stage0(x, dego, degi)

    agg1h = _sc_edge_scatter_esplit(xn, se_e, zdeg)[:, :N]

    q, k, v = _stagea(agg1h, degii, W1, b1.reshape(1, HID), Wq, bq.reshape(1, HID),
                      Wk, bk.reshape(1, HID), Wv, bv.reshape(1, HID))

    qf = jnp.fft.rfft(q.T, axis=-1)
    kf = jnp.fft.rfft(k.T, axis=-1)
    mv = jnp.fft.irfft(jnp.sum(qf * jnp.conj(kf), axis=0), n=N) * (1.0 / HID)
    mvp = jnp.concatenate([mv, jnp.full((80 * 128 - N,), -1e30, jnp.float32)]).reshape(80, 128)
    w128, d128 = _topk(mvp)

    v2 = jnp.concatenate([v, v], axis=0)
    h2n = _delayagg(d128[0, :TOPK], w128[0, :TOPK], v2, degoi, Wo, bo.reshape(1, HID))

    h2nh = h2n.reshape(N, 2, LW).transpose(1, 0, 2)
    agg2h = _sc_edge_scatter_fsplit(h2nh, se_f, zdeg)[:, :N]
    agg2 = agg2h.transpose(1, 0, 2).reshape(N, HID)

    g = _stageb(agg2, degii, W2, b2.reshape(1, HID), Wt)
    gh = g.reshape(N, 2, LW).transpose(1, 0, 2)
    gs, gd = _sc_er_gather(gh, se_f)

    d4 = edge_d.reshape(_ER_NCH, ER_C, 1)
    out2 = _ermax(gs, gd, d4, bt.reshape(2, 128), flag)
    return out2.reshape(1, HID)


# speedup vs baseline: 14.0816x; 11.3341x over previous
import functools
import math

import numpy as np

import jax
import jax.numpy as jnp
from jax import lax
from jax.experimental import pallas as pl
from jax.experimental.pallas import tpu as pltpu
from jax.experimental.pallas import tpu_sc as plsc

N = 10000
E = 160000
INF = 128
HID = 256
TOPK = int(math.log(N))
NC, NS = 2, 16
NPAD = 10240
RPS = N // NS
EK_F, FS_CH = 80, 125
EK_E, ES_CH = 40, 125
LW = 128
DAG_T = 400
ER_C = 2000
HP = jax.lax.Precision.HIGHEST
F = 100
FCH = 32

_th = 2.0 * np.pi * np.outer(np.arange(F), np.arange(F)) / F
_W1R = jnp.asarray(np.cos(_th), dtype=jnp.float32)
_W1I = jnp.asarray(-np.sin(_th), dtype=jnp.float32)
_tl = 2.0 * np.pi * np.outer(np.arange(F), np.arange(F)) / (F * F)
_TTR = jnp.asarray(np.cos(_tl), dtype=jnp.float32)
_TTI = jnp.asarray(-np.sin(_tl), dtype=jnp.float32)

_MESH = dict(core_axis_name="c", subcore_axis_name="s")


def _sc_degrees(se_f, ones_rows, zdeg):
    @functools.partial(
        pl.kernel,
        mesh=plsc.VectorSubcoreMesh(**_MESH),
        out_type=jax.ShapeDtypeStruct((NC, NPAD, LW), jnp.float32),
        scratch_types=[
            pltpu.VMEM((FS_CH, EK_F), jnp.int32),
            pltpu.VMEM((EK_F, LW), jnp.float32),
            pltpu.VMEM_SHARED((NPAD, LW), jnp.float32),
            pltpu.SemaphoreType.DMA,
        ],
    )
    def k(se_hbm, ones_hbm, z_hbm, out_hbm, idx_v, ones_v, acc_sh, sem):
        cid = lax.axis_index("c")
        sid = lax.axis_index("s")
        pltpu.sync_copy(se_hbm.at[cid, sid], idx_v)
        pltpu.sync_copy(ones_hbm, ones_v)

        @pl.when(sid == 0)
        def _():
            pltpu.sync_copy(z_hbm, acc_sh)

        plsc.subcore_barrier()

        def body(j, carry):
            pltpu.sync_copy(ones_v, acc_sh.at[idx_v.at[j]], add=True)
            return carry

        lax.fori_loop(0, FS_CH, body, 0)
        plsc.subcore_barrier()
        sl = pl.ds(sid * (NPAD // NS), NPAD // NS)
        pltpu.sync_copy(acc_sh.at[sl], out_hbm.at[cid].at[sl])

    return k(se_f, ones_rows, zdeg)


def _sc_edge_scatter_fsplit(table_h, se_f, zrows):
    @functools.partial(
        pl.kernel,
        mesh=plsc.VectorSubcoreMesh(**_MESH),
        out_type=jax.ShapeDtypeStruct((NC, NPAD, LW), jnp.float32),
        scratch_types=[
            pltpu.VMEM((FS_CH, EK_F), jnp.int32),
            pltpu.VMEM((FS_CH, EK_F), jnp.int32),
            pltpu.VMEM((EK_F, LW), jnp.float32),
            pltpu.VMEM_SHARED((NPAD, LW), jnp.float32),
            pltpu.SemaphoreType.DMA,
        ],
    )
    def k(tab_hbm, se_hbm, z_hbm, out_hbm, sidx_v, didx_v, rows_v, acc_sh, sem):
        cid = lax.axis_index("c")
        sid = lax.axis_index("s")
        pltpu.sync_copy(se_hbm.at[0, sid], sidx_v)
        pltpu.sync_copy(se_hbm.at[1, sid], didx_v)

        @pl.when(sid == 0)
        def _():
            pltpu.sync_copy(z_hbm, acc_sh)

        plsc.subcore_barrier()

        def body(j, carry):
            pltpu.async_copy(tab_hbm.at[cid].at[sidx_v.at[j]], rows_v, sem).wait()
            pltpu.sync_copy(rows_v, acc_sh.at[didx_v.at[j]], add=True)
            return carry

        lax.fori_loop(0, FS_CH, body, 0)
        plsc.subcore_barrier()
        sl = pl.ds(sid * (NPAD // NS), NPAD // NS)
        pltpu.sync_copy(acc_sh.at[sl], out_hbm.at[cid].at[sl])

    return k(table_h, se_f, zrows)


def _sc_edge_scatter_esplit(table_h, se_e, zrows):
    @functools.partial(
        pl.kernel,
        mesh=plsc.VectorSubcoreMesh(**_MESH),
        out_type=jax.ShapeDtypeStruct((NC, NPAD, LW), jnp.float32),
        scratch_types=[
            pltpu.VMEM((ES_CH, EK_E), jnp.int32),
            pltpu.VMEM((ES_CH, EK_E), jnp.int32),
            pltpu.VMEM((EK_E, LW), jnp.float32),
            pltpu.VMEM_SHARED((NPAD, LW), jnp.float32),
            pltpu.SemaphoreType.DMA,
        ],
    )
    def k(tab_hbm, se_hbm, z_hbm, out_hbm, sidx_v, didx_v, rows_v, acc_sh, sem):
        cid = lax.axis_index("c")
        sid = lax.axis_index("s")
        pltpu.sync_copy(se_hbm.at[0, cid, sid], sidx_v)
        pltpu.sync_copy(se_hbm.at[1, cid, sid], didx_v)

        @pl.when(sid == 0)
        def _():
            pltpu.sync_copy(z_hbm, acc_sh)

        plsc.subcore_barrier()

        def body(j, carry):
            pltpu.async_copy(tab_hbm.at[sidx_v.at[j]], rows_v, sem).wait()
            pltpu.sync_copy(rows_v, acc_sh.at[didx_v.at[j]], add=True)
            return carry

        lax.fori_loop(0, ES_CH, body, 0)
        plsc.subcore_barrier()
        sl = pl.ds(sid * (NPAD // NS), NPAD // NS)
        pltpu.sync_copy(acc_sh.at[sl], out_hbm.at[cid].at[sl])

    return k(table_h, se_e, zrows)


def _sc_er_gather(g_h, se_f):
    @functools.partial(
        pl.kernel,
        mesh=plsc.VectorSubcoreMesh(**_MESH),
        out_type=(
            jax.ShapeDtypeStruct((NC, E, LW), jnp.float32),
            jax.ShapeDtypeStruct((NC, E, LW), jnp.float32),
        ),
        scratch_types=[
            pltpu.VMEM((FS_CH, EK_F), jnp.int32),
            pltpu.VMEM((FS_CH, EK_F), jnp.int32),
            pltpu.VMEM((EK_F, LW), jnp.float32),
            pltpu.VMEM((EK_F, LW), jnp.float32),
            pltpu.SemaphoreType.DMA,
        ],
    )
    def k(g_hbm, se_hbm, gs_hbm, gd_hbm, sidx_v, didx_v, ra, rb, sem):
        cid = lax.axis_index("c")
        sid = lax.axis_index("s")
        pltpu.sync_copy(se_hbm.at[0, sid], sidx_v)
        pltpu.sync_copy(se_hbm.at[1, sid], didx_v)

        def body(j, carry):
            base = sid * (E // NS) + j * EK_F
            pltpu.async_copy(g_hbm.at[cid].at[sidx_v.at[j]], ra, sem).wait()
            pltpu.sync_copy(ra, gs_hbm.at[cid].at[pl.ds(base, EK_F)])
            pltpu.async_copy(g_hbm.at[cid].at[didx_v.at[j]], rb, sem).wait()
            pltpu.sync_copy(rb, gd_hbm.at[cid].at[pl.ds(base, EK_F)])
            return carry

        lax.fori_loop(0, FS_CH, body, 0)

    return k(g_h, se_f)


def _fft1_body(xq_ref, xk_ref, wr_ref, wi_ref, tr_ref, ti_ref,
               qr_ref, qi_ref, kr_ref, ki_ref):
    wr, wi = wr_ref[...], wi_ref[...]
    tr, ti = tr_ref[...][None], ti_ref[...][None]
    for x_ref, or_ref, oi_ref in ((xq_ref, qr_ref, qi_ref), (xk_ref, kr_ref, ki_ref)):
        x = x_ref[...]
        ar = jnp.dot(x, wr, precision=HP).reshape(FCH, F, F)
        ai = jnp.dot(x, wi, precision=HP).reshape(FCH, F, F)
        or_ref[...] = (ar * tr - ai * ti).reshape(FCH * F, F)
        oi_ref[...] = (ar * ti + ai * tr).reshape(FCH * F, F)


def _fft1(xq, xk):
    R = FCH * F
    blk = lambda i: (i, 0)
    full = lambda i: (0, 0)
    o = jax.ShapeDtypeStruct((HID * F, F), jnp.float32)
    return pl.pallas_call(
        _fft1_body,
        grid=(HID // FCH,),
        in_specs=[pl.BlockSpec((R, F), blk)] * 2 + [pl.BlockSpec((F, F), full)] * 4,
        out_specs=(pl.BlockSpec((R, F), blk),) * 4,
        out_shape=(o, o, o, o),
    )(xq, xk, _W1R, _W1I, _TTR, _TTI)


def _fft2_body(bqr_ref, bqi_ref, bkr_ref, bki_ref, wr_ref, wi_ref, sr_ref, si_ref):
    wr, wi = wr_ref[...], wi_ref[...]

    def tx(r_ref, i_ref):
        br, bi = r_ref[...], i_ref[...]
        xr = jnp.dot(br, wr, precision=HP) - jnp.dot(bi, wi, precision=HP)
        xi = jnp.dot(br, wi, precision=HP) + jnp.dot(bi, wr, precision=HP)
        return xr.reshape(FCH, F, F), xi.reshape(FCH, F, F)

    xqr, xqi = tx(bqr_ref, bqi_ref)
    xkr, xki = tx(bkr_ref, bki_ref)
    srp = jnp.sum(xqr * xkr + xqi * xki, axis=0)
    sip = jnp.sum(xqi * xkr - xqr * xki, axis=0)

    @pl.when(pl.program_id(0) == 0)
    def _():
        sr_ref[...] = srp
        si_ref[...] = sip

    @pl.when(pl.program_id(0) > 0)
    def _():
        sr_ref[...] += srp
        si_ref[...] += sip


def _fft2(bqr, bqi, bkr, bki):
    R = FCH * F
    blk = lambda i: (i, 0)
    full = lambda i: (0, 0)
    o = jax.ShapeDtypeStruct((F, F), jnp.float32)
    return pl.pallas_call(
        _fft2_body,
        grid=(HID // FCH,),
        in_specs=[pl.BlockSpec((R, F), blk)] * 4 + [pl.BlockSpec((F, F), full)] * 2,
        out_specs=(pl.BlockSpec((F, F), full),) * 2,
        out_shape=(o, o),
    )(bqr, bqi, bkr, bki, _W1R, _W1I)


def _idft_body(sr_ref, si_ref, wr_ref, wi_ref, tr_ref, ti_ref, mv_ref):
    sr, si = sr_ref[...], si_ref[...]
    wr, wi = wr_ref[...], wi_ref[...]
    tr, ti = tr_ref[...], ti_ref[...]
    cr = jnp.dot(sr, wr, precision=HP) + jnp.dot(si, wi, precision=HP)
    ci = jnp.dot(si, wr, precision=HP) - jnp.dot(sr, wi, precision=HP)
    dr = cr * tr + ci * ti
    di = ci * tr - cr * ti
    mr = jnp.dot(wr, dr, precision=HP) + jnp.dot(wi, di, precision=HP)
    mv_ref[...] = mr * (1.0 / (F * F * HID))


def _idft(sr, si):
    return pl.pallas_call(
        _idft_body,
        out_shape=jax.ShapeDtypeStruct((F, F), jnp.float32),
    )(sr, si, _W1R, _W1I, _TTR, _TTI)


def _stage0_body(x_ref, dego_ref, degi_ref, xn_ref, oi_ref, ii_ref, fl_ref):
    oi = lax.rsqrt(jnp.maximum(dego_ref[...], 1.0))
    ii = lax.rsqrt(jnp.maximum(degi_ref[...], 1.0))
    xn_ref[...] = x_ref[...] * oi
    oi_ref[...] = oi
    ii_ref[...] = ii
    fl_ref[...] = jnp.where(jnp.min(degi_ref[...]) < 0.5, 1.0, 0.0).reshape(1, 1)


def _stage0(x, dego, degi):
    return pl.pallas_call(
        _stage0_body,
        out_shape=(
            jax.ShapeDtypeStruct((N, INF), jnp.float32),
            jax.ShapeDtypeStruct((N, 1), jnp.float32),
            jax.ShapeDtypeStruct((N, 1), jnp.float32),
            jax.ShapeDtypeStruct((1, 1), jnp.float32),
        ),
    )(x, dego, degi)


def _stagea_body(agg_ref, ii_ref, w1_ref, b1_ref, wq_ref, bq_ref, wk_ref,
                 bk_ref, wv_ref, bv_ref, q_ref, k_ref, v_ref):
    h = (agg_ref[0] + agg_ref[1]) * ii_ref[...]
    h1 = jax.nn.relu(jnp.dot(h, w1_ref[...], precision=HP) + b1_ref[...])
    q_ref[...] = jnp.dot(h1, wq_ref[...], precision=HP) + bq_ref[...]
    k_ref[...] = jnp.dot(h1, wk_ref[...], precision=HP) + bk_ref[...]
    v_ref[...] = jnp.dot(h1, wv_ref[...], precision=HP) + bv_ref[...]


def _stagea(agg1, degii, W1, b1, Wq, bq, Wk, bk, Wv, bv):
    R = 2000
    full = lambda i: (0, 0)
    blk = lambda i: (i, 0)
    return pl.pallas_call(
        _stagea_body,
        grid=(N // R,),
        in_specs=[
            pl.BlockSpec((NC, R, INF), lambda i: (0, i, 0)),
            pl.BlockSpec((R, 1), blk),
            pl.BlockSpec((INF, HID), full),
            pl.BlockSpec((1, HID), full),
            pl.BlockSpec((HID, HID), full),
            pl.BlockSpec((1, HID), full),
            pl.BlockSpec((HID, HID), full),
            pl.BlockSpec((1, HID), full),
            pl.BlockSpec((HID, HID), full),
            pl.BlockSpec((1, HID), full),
        ],
        out_specs=(
            pl.BlockSpec((R, HID), blk),
            pl.BlockSpec((R, HID), blk),
            pl.BlockSpec((R, HID), blk),
        ),
        out_shape=(
            jax.ShapeDtypeStruct((N, HID), jnp.float32),
            jax.ShapeDtypeStruct((N, HID), jnp.float32),
            jax.ShapeDtypeStruct((N, HID), jnp.float32),
        ),
    )(agg1, degii, W1, b1, Wq, bq, Wk, bk, Wv, bv)


def _topk_body(mv_ref, w_ref, d_ref):
    arr = mv_ref[...]
    row = lax.broadcasted_iota(jnp.int32, (80, 128), 0)
    col = lax.broadcasted_iota(jnp.int32, (80, 128), 1)
    flat = row * 128 + col
    vals, idxs = [], []
    for _ in range(TOPK):
        m = jnp.max(arr)
        idx = jnp.min(jnp.where(arr >= m, flat, jnp.int32(2 ** 30)))
        vals.append(m)
        idxs.append(idx)
        arr = jnp.where(flat == idx, -1e30, arr)
    m9 = vals[0]
    es = [jnp.exp(vv - m9) for vv in vals]
    s = es[0]
    for e in es[1:]:
        s = s + e
    lane = lax.broadcasted_iota(jnp.int32, (1, 128), 1)
    w = jnp.zeros((1, 128), jnp.float32)
    dl = jnp.zeros((1, 128), jnp.int32)
    for i in range(TOPK):
        w = jnp.where(lane == i, es[i] / s, w)
        dl = jnp.where(lane == i, idxs[i], dl)
    w_ref[...] = w
    d_ref[...] = dl


def _topk(mvp):
    return pl.pallas_call(
        _topk_body,
        out_shape=(
            jax.ShapeDtypeStruct((1, 128), jnp.float32),
            jax.ShapeDtypeStruct((1, 128), jnp.int32),
        ),
    )(mvp)


def _dagg_body(d_sref, w_sref, v2_ref, oi_ref, wo_ref, bo_ref, out_ref):
    base = pl.program_id(0) * DAG_T

    def shifted(i):
        s = base + d_sref[i]
        a = (s // 8) * 8
        r = s - a
        blk = v2_ref[pl.ds(a, DAG_T + 8), :]
        return pltpu.roll(blk, jnp.mod(-r, DAG_T + 8), 0)[:DAG_T]

    acc = w_sref[0] * shifted(0)
    for i in range(1, TOPK):
        acc = acc + w_sref[i] * shifted(i)
    h = jnp.dot(acc, wo_ref[...], precision=HP) + bo_ref[...]
    out_ref[...] = h * oi_ref[...]


def _delayagg(delays9, w9, v2, degoi, Wo, bo):
    grid_spec = pltpu.PrefetchScalarGridSpec(
        num_scalar_prefetch=2,
        grid=(N // DAG_T,),
        in_specs=[
            pl.BlockSpec((2 * N, HID), lambda i, d, w: (0, 0)),
            pl.BlockSpec((DAG_T, 1), lambda i, d, w: (i, 0)),
            pl.BlockSpec((HID, HID), lambda i, d, w: (0, 0)),
            pl.BlockSpec((1, HID), lambda i, d, w: (0, 0)),
        ],
        out_specs=pl.BlockSpec((DAG_T, HID), lambda i, d, w: (i, 0)),
    )
    return pl.pallas_call(
        _dagg_body,
        grid_spec=grid_spec,
        out_shape=jax.ShapeDtypeStruct((N, HID), jnp.float32),
    )(delays9, w9, v2, degoi, Wo, bo)


def _stageb_body(agg_ref, ii_ref, w2_ref, b2_ref, wt_ref, g_ref):
    h = agg_ref[...] * ii_ref[...]
    h3 = jax.nn.relu(jnp.dot(h, w2_ref[...], precision=HP) + b2_ref[...])
    g_ref[...] = jnp.dot(h3, wt_ref[...], precision=HP)


def _stageb(agg2, degii, W2, b2, Wt):
    R = 2000
    full = lambda i: (0, 0)
    blk = lambda i: (i, 0)
    return pl.pallas_call(
        _stageb_body,
        grid=(N // R,),
        in_specs=[
            pl.BlockSpec((R, HID), blk),
            pl.BlockSpec((R, 1), blk),
            pl.BlockSpec((HID, HID), full),
            pl.BlockSpec((1, HID), full),
            pl.BlockSpec((HID, HID), full),
        ],
        out_specs=pl.BlockSpec((R, HID), blk),
        out_shape=jax.ShapeDtypeStruct((N, HID), jnp.float32),
    )(agg2, degii, W2, b2, Wt)


_ER_NCH = E // ER_C


def _ermax_body(gs_ref, gd_ref, d_ref, bt_ref, fl_ref, out_ref):
    j = pl.program_id(0)

    @pl.when(j == 0)
    def _():
        out_ref[...] = jnp.full((NC, 128), -jnp.inf, jnp.float32)

    msg = d_ref[0] * (gs_ref[...] + gd_ref[...])
    cur = jnp.maximum(out_ref[...], jnp.max(msg, axis=1))

    @pl.when(j < _ER_NCH - 1)
    def _():
        out_ref[...] = cur

    @pl.when(j == _ER_NCH - 1)
    def _():
        mm = cur + bt_ref[...]
        out_ref[...] = jnp.where(fl_ref[0, 0] > 0.5, jnp.maximum(mm, 0.0), mm)


def _ermax(gs, gd, d4, btr, flag):
    return pl.pallas_call(
        _ermax_body,
        grid=(_ER_NCH,),
        in_specs=[
            pl.BlockSpec((NC, ER_C, INF), lambda j: (0, j, 0)),
            pl.BlockSpec((NC, ER_C, INF), lambda j: (0, j, 0)),
            pl.BlockSpec((1, ER_C, 1), lambda j: (j, 0, 0)),
            pl.BlockSpec((NC, 128), lambda j: (0, 0)),
            pl.BlockSpec((1, 1), lambda j: (0, 0)),
        ],
        out_specs=pl.BlockSpec((NC, 128), lambda j: (0, 0)),
        out_shape=jax.ShapeDtypeStruct((NC, 128), jnp.float32),
    )(gs, gd, d4, btr, flag)


def kernel(x, edge_index, edge_d, W1, b1, Wq, bq, Wk, bk, Wv, bv, Wo, bo, W2, b2, Wt, bt):
    se = jnp.stack([edge_index[0], edge_index[1]])
    se_f = se.reshape(2, NS, FS_CH, EK_F)
    se_e = se.reshape(2, NC, NS, ES_CH, EK_E)
    ones_rows = jnp.ones((EK_F, LW), jnp.float32)
    zdeg = jnp.zeros((NPAD, LW), jnp.float32)

    degs = _sc_degrees(se_f, ones_rows, zdeg)
    dego = degs[0, :N, 0].reshape(N, 1)
    degi = degs[1, :N, 0].reshape(N, 1)

    xn, degoi, degii, flag = _stage0(x, dego, degi)

    agg1h = _sc_edge_scatter_esplit(xn, se_e, zdeg)[:, :N]

    q, k, v = _stagea(agg1h, degii, W1, b1.reshape(1, HID), Wq, bq.reshape(1, HID),
                      Wk, bk.reshape(1, HID), Wv, bv.reshape(1, HID))

    xq = q.reshape(F, F, HID).transpose(2, 1, 0).reshape(HID * F, F)
    xk = k.reshape(F, F, HID).transpose(2, 1, 0).reshape(HID * F, F)
    bqr, bqi, bkr, bki = _fft1(xq, xk)
    tp = lambda b: b.reshape(HID, F, F).transpose(0, 2, 1).reshape(HID * F, F)
    sr, si = _fft2(tp(bqr), tp(bqi), tp(bkr), tp(bki))
    mv = _idft(sr, si).reshape(N)
    mvp = jnp.concatenate([mv, jnp.full((80 * 128 - N,), -1e30, jnp.float32)]).reshape(80, 128)
    w128, d128 = _topk(mvp)

    v2 = jnp.concatenate([v, v], axis=0)
    h2n = _delayagg(d128[0, :TOPK], w128[0, :TOPK], v2, degoi, Wo, bo.reshape(1, HID))

    h2nh = h2n.reshape(N, 2, LW).transpose(1, 0, 2)
    agg2h = _sc_edge_scatter_fsplit(h2nh, se_f, zdeg)[:, :N]
    agg2 = agg2h.transpose(1, 0, 2).reshape(N, HID)

    g = _stageb(agg2, degii, W2, b2.reshape(1, HID), Wt)
    gh = g.reshape(N, 2, LW).transpose(1, 0, 2)
    gs, gd = _sc_er_gather(gh, se_f)

    d4 = edge_d.reshape(_ER_NCH, ER_C, 1)
    out2 = _ermax(gs, gd, d4, bt.reshape(2, 128), flag)
    return out2.reshape(1, HID)
